# baseline (device time: 49108 ns/iter reference)
import jax
import jax.numpy as jnp
from jax import lax
from jax.experimental import pallas as pl
from jax.experimental.pallas import tpu as pltpu

B, S, H, Dh, Dr = 2, 256, 16, 64, 32
D = 1024
DC = 64
SQ = S // 2
SCALE = (Dh + Dr) ** -0.5


def kernel(x, Wdkv, Wuk, Wuv, Wq, Wqr, Wkr, Wo):
    def body(x_ref, wdkv_ref, wuk_ref, wuv_ref, wq_ref, wqr_ref, wkr_ref,
             wo_ref, out_ref, c_send, c_recv, wuk_recv, wuv_recv, o_scr,
             send_sems, recv_sems):
        my_x = lax.axis_index("x")
        my_y = lax.axis_index("y")
        my_z = lax.axis_index("z")
        y_nbr = (my_x, 1 - my_y, my_z)
        x_nbr = (1 - my_x, my_y, my_z)
        row0 = my_x * SQ

        for b in range(B):
            c_send[b] = jnp.dot(x_ref[b], wdkv_ref[...],
                                preferred_element_type=jnp.float32)

        barrier_sem = pltpu.get_barrier_semaphore()
        for nbr in (y_nbr, x_nbr):
            pl.semaphore_signal(barrier_sem, inc=1, device_id=nbr,
                                device_id_type=pl.DeviceIdType.MESH)
        pl.semaphore_wait(barrier_sem, 2)

        y_rdmas = []
        for i, (src, dst) in enumerate(
                [(c_send, c_recv), (wuk_ref, wuk_recv), (wuv_ref, wuv_recv)]):
            r = pltpu.make_async_remote_copy(
                src_ref=src, dst_ref=dst,
                send_sem=send_sems.at[i], recv_sem=recv_sems.at[i],
                device_id=y_nbr, device_id_type=pl.DeviceIdType.MESH)
            r.start()
            y_rdmas.append(r)

        qs, qrs, krs = [], [], []
        for b in range(B):
            xb = x_ref[b]
            xq = x_ref[b, pl.ds(row0, SQ), :]
            qs.append(jnp.dot(xq, wq_ref[...],
                              preferred_element_type=jnp.float32))
            qrs.append(jnp.dot(xq, wqr_ref[...],
                               preferred_element_type=jnp.float32))
            krs.append(jnp.dot(xb, wkr_ref[...],
                               preferred_element_type=jnp.float32))

        for r in y_rdmas:
            r.wait()

        x_rdmas = []
        for b in range(B):
            qb, qrb, krb = qs[b], qrs[b], krs[b]
            kb = (jnp.dot(c_send[b], wuk_ref[...],
                          preferred_element_type=jnp.float32)
                  + jnp.dot(c_recv[b], wuk_recv[...],
                            preferred_element_type=jnp.float32))
            vb = (jnp.dot(c_send[b], wuv_ref[...],
                          preferred_element_type=jnp.float32)
                  + jnp.dot(c_recv[b], wuv_recv[...],
                            preferred_element_type=jnp.float32))
            for h in range(H):
                qh = qb[:, h * Dh:(h + 1) * Dh]
                kh = kb[:, h * Dh:(h + 1) * Dh]
                qrh = qrb[:, h * Dr:(h + 1) * Dr]
                s1 = lax.dot_general(qh, kh, (((1,), (1,)), ((), ())),
                                     preferred_element_type=jnp.float32)
                s2 = lax.dot_general(qrh, krb, (((1,), (1,)), ((), ())),
                                     preferred_element_type=jnp.float32)
                sc = (s1 + s2) * SCALE
                m = jnp.max(sc, axis=-1, keepdims=True)
                p = jnp.exp(sc - m)
                p = p / jnp.sum(p, axis=-1, keepdims=True)
                vh = vb[:, h * Dh:(h + 1) * Dh]
                o_scr[:, h * Dh:(h + 1) * Dh] = jnp.dot(
                    p, vh, preferred_element_type=jnp.float32)
            out_ref[b, pl.ds(row0, SQ), :] = jnp.dot(
                o_scr[...], wo_ref[...], preferred_element_type=jnp.float32)
            r = pltpu.make_async_remote_copy(
                src_ref=out_ref.at[b, pl.ds(row0, SQ), :],
                dst_ref=out_ref.at[b, pl.ds(row0, SQ), :],
                send_sem=send_sems.at[3 + b], recv_sem=recv_sems.at[3 + b],
                device_id=x_nbr, device_id_type=pl.DeviceIdType.MESH)
            r.start()
            x_rdmas.append(r)

        for r in x_rdmas:
            r.wait()

    return pl.pallas_call(
        body,
        out_shape=jax.ShapeDtypeStruct((B, S, D), jnp.float32),
        in_specs=[pl.BlockSpec(memory_space=pltpu.VMEM)] * 8,
        out_specs=pl.BlockSpec(memory_space=pltpu.VMEM),
        scratch_shapes=[
            pltpu.VMEM((B, S, DC), jnp.float32),
            pltpu.VMEM((B, S, DC), jnp.float32),
            pltpu.VMEM((DC, D), jnp.float32),
            pltpu.VMEM((DC, D), jnp.float32),
            pltpu.VMEM((SQ, D), jnp.float32),
            pltpu.SemaphoreType.DMA((5,)),
            pltpu.SemaphoreType.DMA((5,)),
        ],
        compiler_params=pltpu.CompilerParams(collective_id=0),
    )(x, Wdkv, Wuk, Wuv, Wq, Wqr, Wkr, Wo)


# device time: 29796 ns/iter; 1.6481x vs baseline; 1.6481x over previous
import jax
import jax.numpy as jnp
from jax import lax
from jax.experimental import pallas as pl
from jax.experimental.pallas import tpu as pltpu

B, S, H, Dh, Dr = 2, 256, 16, 64, 32
D = 1024
DC = 64
SCALE = (Dh + Dr) ** -0.5


def kernel(x, Wdkv, Wuk, Wuv, Wq, Wqr, Wkr, Wo):
    def body(x_ref, wdkv_ref, wuk_ref, wuv_ref, wq_ref, wqr_ref, wkr_ref,
             wo_ref, out_hbm, c_send, c_recv, wuk_send, wuk_recv, wuv_send,
             wuv_recv, o_scr, out_v, send_sems, recv_sems, store_sems):
        my_x = lax.axis_index("x")
        my_y = lax.axis_index("y")
        my_z = lax.axis_index("z")
        nbr = (my_x, 1 - my_y, my_z)
        bf = jnp.bfloat16

        wuk_send[...] = wuk_ref[...].astype(bf)
        wuv_send[...] = wuv_ref[...].astype(bf)
        for b in range(B):
            c_send[b] = jnp.dot(x_ref[b], wdkv_ref[...],
                                preferred_element_type=jnp.float32).astype(bf)

        barrier_sem = pltpu.get_barrier_semaphore()
        pl.semaphore_signal(barrier_sem, inc=1, device_id=nbr,
                            device_id_type=pl.DeviceIdType.MESH)
        pl.semaphore_wait(barrier_sem, 1)

        rdmas = []
        for i, (src, dst) in enumerate(
                [(c_send, c_recv), (wuk_send, wuk_recv),
                 (wuv_send, wuv_recv)]):
            r = pltpu.make_async_remote_copy(
                src_ref=src, dst_ref=dst,
                send_sem=send_sems.at[i], recv_sem=recv_sems.at[i],
                device_id=nbr, device_id_type=pl.DeviceIdType.MESH)
            r.start()
            rdmas.append(r)

        qs, qrs, krs = [], [], []
        for b in range(B):
            xb = x_ref[b]
            qs.append(jnp.dot(xb, wq_ref[...],
                              preferred_element_type=jnp.float32))
            qrs.append(jnp.dot(xb, wqr_ref[...],
                               preferred_element_type=jnp.float32))
            krs.append(jnp.dot(xb, wkr_ref[...],
                               preferred_element_type=jnp.float32))

        for r in rdmas:
            r.wait()

        ones_col = jnp.ones((S, 1), dtype=jnp.float32)
        stores = []
        for b in range(B):
            qb, qrb, krb = qs[b] * SCALE, qrs[b] * SCALE, krs[b]
            kb = (jnp.dot(c_send[b], wuk_send[...],
                          preferred_element_type=jnp.float32)
                  + jnp.dot(c_recv[b], wuk_recv[...],
                            preferred_element_type=jnp.float32))
            vb = (jnp.dot(c_send[b], wuv_send[...],
                          preferred_element_type=jnp.float32)
                  + jnp.dot(c_recv[b], wuv_recv[...],
                            preferred_element_type=jnp.float32))
            for h in range(H):
                qh = qb[:, h * Dh:(h + 1) * Dh]
                kh = kb[:, h * Dh:(h + 1) * Dh]
                qrh = qrb[:, h * Dr:(h + 1) * Dr]
                s1 = lax.dot_general(qh, kh, (((1,), (1,)), ((), ())),
                                     preferred_element_type=jnp.float32)
                s2 = lax.dot_general(qrh, krb, (((1,), (1,)), ((), ())),
                                     preferred_element_type=jnp.float32)
                p = jnp.exp(s1 + s2)
                rs = jnp.dot(p, ones_col, preferred_element_type=jnp.float32)
                vh = vb[:, h * Dh:(h + 1) * Dh]
                o = jnp.dot(p, vh, preferred_element_type=jnp.float32)
                o_scr[:, h * Dh:(h + 1) * Dh] = o * (1.0 / rs)
            out_v[b] = jnp.dot(o_scr[...], wo_ref[...],
                               preferred_element_type=jnp.float32)
            cp = pltpu.make_async_copy(out_v.at[b], out_hbm.at[b],
                                       store_sems.at[b])
            cp.start()
            stores.append(cp)

        for cp in stores:
            cp.wait()

    vmem = pl.BlockSpec(memory_space=pltpu.VMEM)
    return pl.pallas_call(
        body,
        out_shape=jax.ShapeDtypeStruct((B, S, D), jnp.float32),
        in_specs=[vmem] * 8,
        out_specs=pl.BlockSpec(memory_space=pl.ANY),
        scratch_shapes=[
            pltpu.VMEM((B, S, DC), jnp.bfloat16),
            pltpu.VMEM((B, S, DC), jnp.bfloat16),
            pltpu.VMEM((DC, D), jnp.bfloat16),
            pltpu.VMEM((DC, D), jnp.bfloat16),
            pltpu.VMEM((DC, D), jnp.bfloat16),
            pltpu.VMEM((DC, D), jnp.bfloat16),
            pltpu.VMEM((S, D), jnp.float32),
            pltpu.VMEM((B, S, D), jnp.float32),
            pltpu.SemaphoreType.DMA((3,)),
            pltpu.SemaphoreType.DMA((3,)),
            pltpu.SemaphoreType.DMA((B,)),
        ],
        compiler_params=pltpu.CompilerParams(collective_id=0),
    )(x, Wdkv, Wuk, Wuv, Wq, Wqr, Wkr, Wo)
